# baseline (speedup 1.0000x reference)
import functools

import jax
import jax.numpy as jnp
import numpy as np
from jax import lax
from jax.experimental import pallas as pl
from jax.experimental.pallas import tpu as pltpu
from jax.experimental.pallas import tpu_sc as plsc

N_ATOM = 10000
N_MOTIF = 2000
N_ALL = N_ATOM + N_MOTIF
NHID = 128
NLAYER = 5
NGRAPH = 256

NC = 2
NS = 16
NW = NC * NS
CH = 80
LANES = 16

_MESH = plsc.VectorSubcoreMesh(core_axis_name="c", subcore_axis_name="s")


def _add_relu_rows_packed(dst_ref, a_ref, e_ref, n_rows):
    msk = jnp.int32(-65536)
    def body(r, carry):
        for g in range(NHID // 32):
            w = e_ref[r, pl.ds(g * 16, 16)]
            lo = lax.bitcast_convert_type(w << 16, jnp.float32)
            hi = lax.bitcast_convert_type(w & msk, jnp.float32)
            s0 = pl.ds(g * 32, LANES)
            s1 = pl.ds(g * 32 + 16, LANES)
            dst_ref[r, s0] = jnp.maximum(a_ref[r, s0] + lo, 0.0)
            dst_ref[r, s1] = jnp.maximum(a_ref[r, s1] + hi, 0.0)
        return carry
    lax.fori_loop(0, n_rows, body, 0)


SUP = 10


def _make_segsum(n_edges, n_acc, with_e):
    n_chunks = n_edges // CH
    n_sup = n_chunks // SUP
    assert n_sup * SUP * CH == n_edges
    BR = 16
    n_blk = n_acc // BR

    scratch = [
        pltpu.VMEM((SUP, CH), jnp.int32),
        pltpu.VMEM((SUP, CH), jnp.int32),
        pltpu.VMEM((CH, NHID), jnp.float32),
        pltpu.VMEM((CH, NHID), jnp.float32),
        pltpu.VMEM((CH, NHID // 2), jnp.int32),
        pltpu.VMEM_SHARED((n_acc, NHID), jnp.float32),
        pltpu.SemaphoreType.DMA,
        pltpu.SemaphoreType.DMA,
        pltpu.SemaphoreType.DMA,
        pltpu.SemaphoreType.DMA,
        pltpu.SemaphoreType.DMA,
    ]

    def impl(x_hbm, src_hbm, dst_hbm, e_hbm, zero_hbm, out_hbm,
             idxg, idxs, r0, r1, ebuf, acc_sh,
             g0, g1, es0, s0, s1):
        rbuf = (r0, r1)
        gsem = (g0, g1)
        esem = es0
        ssem = (s0, s1)
        cid = lax.axis_index("c")
        sid = lax.axis_index("s")
        wid = cid * NS + sid
        my_blk = (n_blk - sid + NS - 1) // NS

        def zblk(i, carry):
            r = (sid + i * NS) * BR
            pltpu.sync_copy(zero_hbm.at[pl.ds(r, BR)],
                            acc_sh.at[pl.ds(r, BR)])
            return carry
        lax.fori_loop(0, my_blk, zblk, 0)
        plsc.subcore_barrier()

        def sup(i, carry):
            s = wid + i * NW
            base0 = s * (SUP * CH)
            pltpu.sync_copy(src_hbm.at[s], idxg)
            pltpu.sync_copy(dst_hbm.at[s], idxs)
            pltpu.async_copy(x_hbm.at[idxg.at[0]], rbuf[0], gsem[0])
            if with_e:
                pltpu.async_copy(e_hbm.at[pl.ds(base0, CH)], ebuf, esem)
            for k in range(SUP):
                b = k % 2
                nb = (k + 1) % 2
                if k + 1 < SUP:
                    if k >= 1:
                        pltpu.make_async_copy(
                            rbuf[nb], acc_sh.at[idxs.at[k - 1]],
                            ssem[nb]).wait()
                    pltpu.async_copy(x_hbm.at[idxg.at[k + 1]], rbuf[nb],
                                     gsem[nb])
                pltpu.make_async_copy(x_hbm.at[idxg.at[k]], rbuf[b],
                                      gsem[b]).wait()
                if with_e:
                    pltpu.make_async_copy(
                        e_hbm.at[pl.ds(base0 + k * CH, CH)], ebuf,
                        esem).wait()
                    _add_relu_rows_packed(rbuf[b], rbuf[b], ebuf, CH)
                    if k + 1 < SUP:
                        pltpu.async_copy(
                            e_hbm.at[pl.ds(base0 + (k + 1) * CH, CH)],
                            ebuf, esem)
                pltpu.async_copy(rbuf[b], acc_sh.at[idxs.at[k]], ssem[b],
                                 add=True)
            pltpu.make_async_copy(rbuf[0], acc_sh.at[idxs.at[SUP - 2]],
                                  ssem[0]).wait()
            pltpu.make_async_copy(rbuf[1], acc_sh.at[idxs.at[SUP - 1]],
                                  ssem[1]).wait()
            return carry

        my_sup = (n_sup - wid + NW - 1) // NW
        lax.fori_loop(0, my_sup, sup, 0)
        plsc.subcore_barrier()

        def oblk(i, carry):
            r = (sid + i * NS) * BR
            pltpu.sync_copy(acc_sh.at[pl.ds(r, BR)],
                            out_hbm.at[cid, pl.ds(r, BR)])
            return carry
        lax.fori_loop(0, my_blk, oblk, 0)

    if with_e:
        def body(x_hbm, src_hbm, dst_hbm, e_hbm, zero_hbm, out_hbm,
                 idxg, idxs, r0, r1, ebuf, acc_sh, g0, g1, es0, s0, s1):
            impl(x_hbm, src_hbm, dst_hbm, e_hbm, zero_hbm, out_hbm,
                 idxg, idxs, r0, r1, ebuf, acc_sh, g0, g1, es0, s0, s1)
    else:
        def body(x_hbm, src_hbm, dst_hbm, zero_hbm, out_hbm,
                 idxg, idxs, r0, r1, ebuf, acc_sh, g0, g1, es0, s0, s1):
            impl(x_hbm, src_hbm, dst_hbm, None, zero_hbm, out_hbm,
                 idxg, idxs, r0, r1, ebuf, acc_sh, g0, g1, es0, s0, s1)

    out_t = jax.ShapeDtypeStruct((NC, n_acc, NHID), jnp.float32)
    return pl.kernel(body, out_type=out_t, mesh=_MESH, scratch_types=scratch,
                     name=f"sc_segsum_e{n_edges}_a{n_acc}_{int(with_e)}")


def _tc_embedsum(feat, tab, blk, out_dtype=jnp.float32):
    n, n_feat = feat.shape
    vp = tab.shape[0]

    def body(f_ref, t_ref, o_ref):
        f = f_ref[...]
        cols = lax.broadcasted_iota(jnp.int32, (blk, vp), 1)
        oh = (cols == f[:, 0:1]).astype(jnp.float32)
        for j in range(1, n_feat):
            oh = oh + (cols == f[:, j:j + 1]).astype(jnp.float32)
        o_ref[...] = jnp.dot(oh, t_ref[...],
                             preferred_element_type=jnp.float32
                             ).astype(out_dtype)

    return pl.pallas_call(
        body,
        grid=(n // blk,),
        in_specs=[
            pl.BlockSpec((blk, n_feat), lambda i: (i, 0)),
            pl.BlockSpec((vp, NHID), lambda i: (0, 0)),
        ],
        out_specs=pl.BlockSpec((blk, NHID), lambda i: (i, 0)),
        out_shape=jax.ShapeDtypeStruct((n, NHID), out_dtype),
    )(feat, tab)


_BLK = 1000


def _tc_layer_body(x_ref, p_ref, w_ref, b_ref, o_ref):
    x = x_ref[...]
    h = x + p_ref[0] + p_ref[1]
    y = jnp.dot(h, w_ref[0], preferred_element_type=jnp.float32) + b_ref[0, 0]
    o_ref[...] = jnp.maximum(y, 0.0) + x


def _tc_layer(x, part, w2, b2):
    n_atom_blk = N_ATOM // _BLK
    grid = N_ALL // _BLK

    def wsel(i):
        s = jnp.where(i >= n_atom_blk, 1, 0)
        return (s, 0, 0)

    def bsel(i):
        s = jnp.where(i >= n_atom_blk, 1, 0)
        return (s, 0, 0)

    return pl.pallas_call(
        _tc_layer_body,
        grid=(grid,),
        in_specs=[
            pl.BlockSpec((_BLK, NHID), lambda i: (i, 0)),
            pl.BlockSpec((NC, _BLK, NHID), lambda i: (0, i, 0)),
            pl.BlockSpec((1, NHID, NHID), wsel),
            pl.BlockSpec((1, 1, NHID), bsel),
        ],
        out_specs=pl.BlockSpec((_BLK, NHID), lambda i: (i, 0)),
        out_shape=jax.ShapeDtypeStruct((N_ALL, NHID), jnp.float32),
    )(x, part, w2, b2)


def _tc_deepset_body(p_ref, w_ref, b_ref, o_ref):
    h = p_ref[0] + p_ref[1]
    y = jnp.dot(h, w_ref[...], preferred_element_type=jnp.float32) + b_ref[...]
    o_ref[...] = jnp.maximum(y, 0.0)


def _tc_deepset(part, w, b):
    grid = N_MOTIF // _BLK
    return pl.pallas_call(
        _tc_deepset_body,
        grid=(grid,),
        in_specs=[
            pl.BlockSpec((NC, _BLK, NHID), lambda i: (0, i, 0)),
            pl.BlockSpec((NHID, NHID), lambda i: (0, 0)),
            pl.BlockSpec((NHID,), lambda i: (0,)),
        ],
        out_specs=pl.BlockSpec((_BLK, NHID), lambda i: (i, 0)),
        out_shape=jax.ShapeDtypeStruct((N_MOTIF, NHID), jnp.float32),
    )(part, w, b)


def kernel(x_atom_feat, x_motif_feat, motif_atoms, motif_atoms_map,
           ei_aa, ea_aa, ei_am, ea_am, ei_ma, ea_ma, ei_mm, ea_mm,
           batch_atom, batch_motif, atom_tables, motif_table,
           bond_aa_tables, am_table, ma_table, mm_table,
           deepset_W, deepset_b, Wa, ba, Wm, bm):
    i32 = jnp.int32

    atab = jnp.concatenate(atom_tables, axis=0)
    a_offs = jnp.array([0, 119, 124, 136, 148, 158, 164, 170, 172], i32)
    feat_a = x_atom_feat.astype(i32) + a_offs[None, :]
    atab = jnp.pad(atab, ((0, 2), (0, 0)))

    btab = jnp.concatenate(
        [bond_aa_tables[0], bond_aa_tables[1], bond_aa_tables[2],
         am_table, ma_table, mm_table,
         jnp.zeros((1, NHID), jnp.float32)], axis=0)
    ZB = 56
    b_offs = jnp.array([0, 22, 28], i32)
    fa = ea_aa.astype(i32) + b_offs[None, :]
    f_ma = jnp.stack([32 + ea_ma.astype(i32),
                      jnp.full_like(ea_ma, ZB, i32),
                      jnp.full_like(ea_ma, ZB, i32)], axis=1)
    f_mm = jnp.stack([34 + ea_mm.astype(i32),
                      jnp.full_like(ea_mm, ZB, i32),
                      jnp.full_like(ea_mm, ZB, i32)], axis=1)
    f_am = jnp.stack([30 + ea_am.astype(i32),
                      jnp.full_like(ea_am, ZB, i32),
                      jnp.full_like(ea_am, ZB, i32)], axis=1)
    featb = jnp.concatenate([fa, f_ma, f_mm, f_am], axis=0)
    n_edges = featb.shape[0]
    btab = jnp.pad(btab, ((0, 7), (0, 0)))

    src_all = jnp.concatenate([
        ei_aa[0].astype(i32),
        N_ATOM + ei_ma[0].astype(i32),
        N_ATOM + ei_mm[0].astype(i32),
        ei_am[0].astype(i32)])
    dst_all = jnp.concatenate([
        ei_aa[1].astype(i32),
        ei_ma[1].astype(i32),
        N_ATOM + ei_mm[1].astype(i32),
        N_ATOM + ei_am[1].astype(i32)])

    zeros_all = jnp.zeros((N_ALL, NHID), jnp.float32)
    zeros_m = jnp.zeros((N_MOTIF, NHID), jnp.float32)

    xa0 = _tc_embedsum(feat_a, atab, 2000)
    perm = np.stack([np.arange(16), np.arange(16) + 16], 1).reshape(-1)
    perm = (np.arange(0, NHID, 32)[:, None] + perm[None, :]).reshape(-1)
    e_bf = _tc_embedsum(featb, btab[:, perm], 2000, jnp.bfloat16)
    e_all = lax.bitcast_convert_type(
        e_bf.reshape(n_edges, NHID // 2, 2), i32)

    def sup3(a):
        return a.astype(i32).reshape(-1, SUP, CH)

    seg_pool = _make_segsum(motif_atoms.shape[0], N_MOTIF, with_e=False)
    pooled = seg_pool(xa0, sup3(motif_atoms), sup3(motif_atoms_map),
                      zeros_m)
    xm0 = _tc_deepset(pooled, deepset_W, deepset_b)

    x = jnp.concatenate([xa0, xm0], axis=0)

    seg_layer = _make_segsum(n_edges, N_ALL, with_e=True)
    src3 = sup3(src_all)
    dst3 = sup3(dst_all)
    stages = [x]
    for l in range(NLAYER):
        part = seg_layer(x, src3, dst3, e_all, zeros_all)
        w2 = jnp.stack([Wa[l], Wm[l]])
        b2 = jnp.stack([ba[l], bm[l]]).reshape(NC, 1, NHID)
        x = _tc_layer(x, part, w2, b2)
        stages.append(x)

    x6 = jnp.concatenate(stages, axis=0)
    batch_all = jnp.concatenate([batch_atom.astype(i32),
                                 NGRAPH + batch_motif.astype(i32)])
    n_pool = 6 * N_ALL
    src_pool = jnp.arange(n_pool, dtype=i32)
    dst_pool = (2 * NGRAPH * jnp.arange(6, dtype=i32)[:, None]
                + batch_all[None, :]).reshape(-1)
    seg_final = _make_segsum(n_pool, 6 * 2 * NGRAPH, with_e=False)
    pp = seg_final(x6, sup3(src_pool), sup3(dst_pool),
                   jnp.zeros((6 * 2 * NGRAPH, NHID), jnp.float32))
    p = (pp[0] + pp[1]).reshape(6, 2 * NGRAPH, NHID)
    atom_embs = p[:, :NGRAPH].transpose(1, 0, 2).reshape(NGRAPH, 6 * NHID)
    motif_embs = p[:, NGRAPH:].transpose(1, 0, 2).reshape(NGRAPH, 6 * NHID)
    return (atom_embs, motif_embs)


# speedup vs baseline: 1.3912x; 1.3912x over previous
import functools

import jax
import jax.numpy as jnp
import numpy as np
from jax import lax
from jax.experimental import pallas as pl
from jax.experimental.pallas import tpu as pltpu
from jax.experimental.pallas import tpu_sc as plsc

N_ATOM = 10000
N_MOTIF = 2000
N_ALL = N_ATOM + N_MOTIF
NHID = 128
NLAYER = 5
NGRAPH = 256

NC = 2
NS = 16
NW = NC * NS
CH = 80
LANES = 16

_MESH = plsc.VectorSubcoreMesh(core_axis_name="c", subcore_axis_name="s")


def _add_relu_rows_packed(dst_ref, a_ref, e_ref, n_rows):
    msk = jnp.int32(-65536)
    def body(r, carry):
        for g in range(NHID // 32):
            w = e_ref[r, pl.ds(g * 16, 16)]
            lo = lax.bitcast_convert_type(w << 16, jnp.float32)
            hi = lax.bitcast_convert_type(w & msk, jnp.float32)
            s0 = pl.ds(g * 16, LANES)
            s1 = pl.ds(NHID // 2 + g * 16, LANES)
            dst_ref[r, s0] = jnp.maximum(a_ref[r, s0] + lo, 0.0)
            dst_ref[r, s1] = jnp.maximum(a_ref[r, s1] + hi, 0.0)
        return carry
    lax.fori_loop(0, n_rows, body, 0)


SUP = 10


def _make_segsum(n_edges, n_acc, with_e):
    n_chunks = n_edges // CH
    n_sup = n_chunks // SUP
    assert n_sup * SUP * CH == n_edges
    BR = 16
    n_blk = n_acc // BR

    scratch = [
        pltpu.VMEM((SUP, CH), jnp.int32),
        pltpu.VMEM((SUP, CH), jnp.int32),
        pltpu.VMEM((CH, NHID), jnp.float32),
        pltpu.VMEM((CH, NHID), jnp.float32),
        pltpu.VMEM((CH, NHID // 2), jnp.int32),
        pltpu.VMEM_SHARED((n_acc, NHID), jnp.float32),
        pltpu.SemaphoreType.DMA,
        pltpu.SemaphoreType.DMA,
        pltpu.SemaphoreType.DMA,
        pltpu.SemaphoreType.DMA,
        pltpu.SemaphoreType.DMA,
    ]

    def impl(x_hbm, src_hbm, dst_hbm, e_hbm, zero_hbm, out_hbm,
             idxg, idxs, r0, r1, ebuf, acc_sh,
             g0, g1, es0, s0, s1):
        rbuf = (r0, r1)
        gsem = (g0, g1)
        esem = es0
        ssem = (s0, s1)
        cid = lax.axis_index("c")
        sid = lax.axis_index("s")
        wid = cid * NS + sid
        my_blk = (n_blk - sid + NS - 1) // NS

        def zblk(i, carry):
            r = (sid + i * NS) * BR
            pltpu.sync_copy(zero_hbm.at[pl.ds(r, BR)],
                            acc_sh.at[pl.ds(r, BR)])
            return carry
        lax.fori_loop(0, my_blk, zblk, 0)
        plsc.subcore_barrier()

        def sup(i, carry):
            s = wid + i * NW
            base0 = s * (SUP * CH)
            pltpu.sync_copy(src_hbm.at[s], idxg)
            pltpu.sync_copy(dst_hbm.at[s], idxs)
            pltpu.async_copy(x_hbm.at[idxg.at[0]], rbuf[0], gsem[0])
            if with_e:
                pltpu.async_copy(e_hbm.at[pl.ds(base0, CH)], ebuf, esem)
            for k in range(SUP):
                b = k % 2
                nb = (k + 1) % 2
                if k + 1 < SUP:
                    if k >= 1:
                        pltpu.make_async_copy(
                            rbuf[nb], acc_sh.at[idxs.at[k - 1]],
                            ssem[nb]).wait()
                    pltpu.async_copy(x_hbm.at[idxg.at[k + 1]], rbuf[nb],
                                     gsem[nb])
                pltpu.make_async_copy(x_hbm.at[idxg.at[k]], rbuf[b],
                                      gsem[b]).wait()
                if with_e:
                    pltpu.make_async_copy(
                        e_hbm.at[pl.ds(base0 + k * CH, CH)], ebuf,
                        esem).wait()
                    _add_relu_rows_packed(rbuf[b], rbuf[b], ebuf, CH)
                    if k + 1 < SUP:
                        pltpu.async_copy(
                            e_hbm.at[pl.ds(base0 + (k + 1) * CH, CH)],
                            ebuf, esem)
                pltpu.async_copy(rbuf[b], acc_sh.at[idxs.at[k]], ssem[b],
                                 add=True)
            pltpu.make_async_copy(rbuf[0], acc_sh.at[idxs.at[SUP - 2]],
                                  ssem[0]).wait()
            pltpu.make_async_copy(rbuf[1], acc_sh.at[idxs.at[SUP - 1]],
                                  ssem[1]).wait()
            return carry

        my_sup = (n_sup - wid + NW - 1) // NW
        lax.fori_loop(0, my_sup, sup, 0)
        plsc.subcore_barrier()

        def oblk(i, carry):
            r = (sid + i * NS) * BR
            pltpu.sync_copy(acc_sh.at[pl.ds(r, BR)],
                            out_hbm.at[cid, pl.ds(r, BR)])
            return carry
        lax.fori_loop(0, my_blk, oblk, 0)

    if with_e:
        def body(x_hbm, src_hbm, dst_hbm, e_hbm, zero_hbm, out_hbm,
                 idxg, idxs, r0, r1, ebuf, acc_sh, g0, g1, es0, s0, s1):
            impl(x_hbm, src_hbm, dst_hbm, e_hbm, zero_hbm, out_hbm,
                 idxg, idxs, r0, r1, ebuf, acc_sh, g0, g1, es0, s0, s1)
    else:
        def body(x_hbm, src_hbm, dst_hbm, zero_hbm, out_hbm,
                 idxg, idxs, r0, r1, ebuf, acc_sh, g0, g1, es0, s0, s1):
            impl(x_hbm, src_hbm, dst_hbm, None, zero_hbm, out_hbm,
                 idxg, idxs, r0, r1, ebuf, acc_sh, g0, g1, es0, s0, s1)

    out_t = jax.ShapeDtypeStruct((NC, n_acc, NHID), jnp.float32)
    return pl.kernel(body, out_type=out_t, mesh=_MESH, scratch_types=scratch,
                     name=f"sc_segsum_e{n_edges}_a{n_acc}_{int(with_e)}")


def _tc_embedsum(feat, tab, blk, out_dtype=jnp.float32):
    n, n_feat = feat.shape
    vp = tab.shape[0]

    pack = out_dtype == jnp.int32
    ow = NHID // 2 if pack else NHID

    def body(f_ref, t_ref, o_ref):
        f = f_ref[...]
        cols = lax.broadcasted_iota(jnp.int32, (blk, vp), 1)
        oh = (cols == f[:, 0:1]).astype(jnp.float32)
        for j in range(1, n_feat):
            oh = oh + (cols == f[:, j:j + 1]).astype(jnp.float32)
        y = jnp.dot(oh, t_ref[...], preferred_element_type=jnp.float32)
        if pack:
            yi = lax.bitcast_convert_type(y, jnp.int32)
            lo = (yi[:, :ow] >> 16) & jnp.int32(0xFFFF)
            hi = yi[:, ow:] & jnp.int32(-65536)
            o_ref[...] = hi | lo
        else:
            o_ref[...] = y.astype(out_dtype)

    return pl.pallas_call(
        body,
        grid=(n // blk,),
        in_specs=[
            pl.BlockSpec((blk, n_feat), lambda i: (i, 0)),
            pl.BlockSpec((vp, NHID), lambda i: (0, 0)),
        ],
        out_specs=pl.BlockSpec((blk, ow), lambda i: (i, 0)),
        out_shape=jax.ShapeDtypeStruct((n, ow), out_dtype),
    )(feat, tab)


_BLK = 1000


def _tc_layer_body(x_ref, p_ref, w_ref, b_ref, o_ref):
    x = x_ref[...]
    h = x + p_ref[0] + p_ref[1]
    y = jnp.dot(h, w_ref[0], preferred_element_type=jnp.float32) + b_ref[0, 0]
    o_ref[...] = jnp.maximum(y, 0.0) + x


def _tc_layer(x, part, w2, b2):
    n_atom_blk = N_ATOM // _BLK
    grid = N_ALL // _BLK

    def wsel(i):
        s = jnp.where(i >= n_atom_blk, 1, 0)
        return (s, 0, 0)

    def bsel(i):
        s = jnp.where(i >= n_atom_blk, 1, 0)
        return (s, 0, 0)

    return pl.pallas_call(
        _tc_layer_body,
        grid=(grid,),
        in_specs=[
            pl.BlockSpec((_BLK, NHID), lambda i: (i, 0)),
            pl.BlockSpec((NC, _BLK, NHID), lambda i: (0, i, 0)),
            pl.BlockSpec((1, NHID, NHID), wsel),
            pl.BlockSpec((1, 1, NHID), bsel),
        ],
        out_specs=pl.BlockSpec((_BLK, NHID), lambda i: (i, 0)),
        out_shape=jax.ShapeDtypeStruct((N_ALL, NHID), jnp.float32),
    )(x, part, w2, b2)


def _tc_deepset_body(p_ref, w_ref, b_ref, o_ref):
    h = p_ref[0] + p_ref[1]
    y = jnp.dot(h, w_ref[...], preferred_element_type=jnp.float32) + b_ref[...]
    o_ref[...] = jnp.maximum(y, 0.0)


def _tc_deepset(part, w, b):
    grid = N_MOTIF // _BLK
    return pl.pallas_call(
        _tc_deepset_body,
        grid=(grid,),
        in_specs=[
            pl.BlockSpec((NC, _BLK, NHID), lambda i: (0, i, 0)),
            pl.BlockSpec((NHID, NHID), lambda i: (0, 0)),
            pl.BlockSpec((NHID,), lambda i: (0,)),
        ],
        out_specs=pl.BlockSpec((_BLK, NHID), lambda i: (i, 0)),
        out_shape=jax.ShapeDtypeStruct((N_MOTIF, NHID), jnp.float32),
    )(part, w, b)


def kernel(x_atom_feat, x_motif_feat, motif_atoms, motif_atoms_map,
           ei_aa, ea_aa, ei_am, ea_am, ei_ma, ea_ma, ei_mm, ea_mm,
           batch_atom, batch_motif, atom_tables, motif_table,
           bond_aa_tables, am_table, ma_table, mm_table,
           deepset_W, deepset_b, Wa, ba, Wm, bm):
    i32 = jnp.int32

    atab = jnp.concatenate(atom_tables, axis=0)
    a_offs = jnp.array([0, 119, 124, 136, 148, 158, 164, 170, 172], i32)
    feat_a = x_atom_feat.astype(i32) + a_offs[None, :]
    atab = jnp.pad(atab, ((0, 2), (0, 0)))

    btab = jnp.concatenate(
        [bond_aa_tables[0], bond_aa_tables[1], bond_aa_tables[2],
         am_table, ma_table, mm_table,
         jnp.zeros((1, NHID), jnp.float32)], axis=0)
    ZB = 56
    b_offs = jnp.array([0, 22, 28], i32)
    fa = ea_aa.astype(i32) + b_offs[None, :]
    f_ma = jnp.stack([32 + ea_ma.astype(i32),
                      jnp.full_like(ea_ma, ZB, i32),
                      jnp.full_like(ea_ma, ZB, i32)], axis=1)
    f_mm = jnp.stack([34 + ea_mm.astype(i32),
                      jnp.full_like(ea_mm, ZB, i32),
                      jnp.full_like(ea_mm, ZB, i32)], axis=1)
    f_am = jnp.stack([30 + ea_am.astype(i32),
                      jnp.full_like(ea_am, ZB, i32),
                      jnp.full_like(ea_am, ZB, i32)], axis=1)
    featb = jnp.concatenate([fa, f_ma, f_mm, f_am], axis=0)
    n_edges = featb.shape[0]
    btab = jnp.pad(btab, ((0, 7), (0, 0)))

    src_all = jnp.concatenate([
        ei_aa[0].astype(i32),
        N_ATOM + ei_ma[0].astype(i32),
        N_ATOM + ei_mm[0].astype(i32),
        ei_am[0].astype(i32)])
    dst_all = jnp.concatenate([
        ei_aa[1].astype(i32),
        ei_ma[1].astype(i32),
        N_ATOM + ei_mm[1].astype(i32),
        N_ATOM + ei_am[1].astype(i32)])

    zeros_all = jnp.zeros((N_ALL, NHID), jnp.float32)
    zeros_m = jnp.zeros((N_MOTIF, NHID), jnp.float32)

    xa0 = _tc_embedsum(feat_a, atab, 2000)
    e_all = _tc_embedsum(featb, btab, 2000, jnp.int32)

    def sup3(a):
        return a.astype(i32).reshape(-1, SUP, CH)

    seg_pool = _make_segsum(motif_atoms.shape[0], N_MOTIF, with_e=False)
    pooled = seg_pool(xa0, sup3(motif_atoms), sup3(motif_atoms_map),
                      zeros_m)
    xm0 = _tc_deepset(pooled, deepset_W, deepset_b)

    x = jnp.concatenate([xa0, xm0], axis=0)

    seg_layer = _make_segsum(n_edges, N_ALL, with_e=True)
    src3 = sup3(src_all)
    dst3 = sup3(dst_all)
    stages = [x]
    for l in range(NLAYER):
        part = seg_layer(x, src3, dst3, e_all, zeros_all)
        w2 = jnp.stack([Wa[l], Wm[l]])
        b2 = jnp.stack([ba[l], bm[l]]).reshape(NC, 1, NHID)
        x = _tc_layer(x, part, w2, b2)
        stages.append(x)

    x6 = jnp.concatenate(stages, axis=0)
    batch_all = jnp.concatenate([batch_atom.astype(i32),
                                 NGRAPH + batch_motif.astype(i32)])
    n_pool = 6 * N_ALL
    src_pool = jnp.arange(n_pool, dtype=i32)
    dst_pool = (2 * NGRAPH * jnp.arange(6, dtype=i32)[:, None]
                + batch_all[None, :]).reshape(-1)
    seg_final = _make_segsum(n_pool, 6 * 2 * NGRAPH, with_e=False)
    pp = seg_final(x6, sup3(src_pool), sup3(dst_pool),
                   jnp.zeros((6 * 2 * NGRAPH, NHID), jnp.float32))
    p = (pp[0] + pp[1]).reshape(6, 2 * NGRAPH, NHID)
    atom_embs = p[:, :NGRAPH].transpose(1, 0, 2).reshape(NGRAPH, 6 * NHID)
    motif_embs = p[:, NGRAPH:].transpose(1, 0, 2).reshape(NGRAPH, 6 * NHID)
    return (atom_embs, motif_embs)
